# baseline (speedup 1.0000x reference)
import jax
import jax.numpy as jnp
from jax import lax
from jax.experimental import pallas as pl
from jax.experimental.pallas import tpu as pltpu, tpu_sc as plsc

V = 100000
F = 26
D = 16
B = 4096
G = 25
IW = 32
SPLIT = 13

NC, NS = 2, 16
NW = NC * NS
B_PER_W = B // NW

PAIRS_A = [(i, j) for i in range(SPLIT) for j in range(i + 1, SPLIT)]
PAIRS_B = [(i, j) for i in range(G) for j in range(max(i + 1, SPLIT), G)]
NPA = len(PAIRS_A)
NPB = len(PAIRS_B)


def _lane():
    return lax.broadcasted_iota(jnp.int32, (D,), 0)


def _fill_pairs(pair_s, pairs):
    for p, (i, j) in enumerate(pairs):
        pair_s[p] = i * IW + j


def _extract_vals(vals_v, vs_s, b):
    v0 = vals_v[b, pl.ds(0, D)]
    v1 = vals_v[b, pl.ds(D, D)]
    for f in range(F):
        vs_s[f] = (v0 if f < D else v1)[f % D]
    return v0, v1


def _pair_sum(rows_v, pair_s, vs_s, bk, npairs, init):
    def pk(k, accs):
        a0, a1 = accs
        pa = pair_s[2 * k]
        i0 = pa // IW
        j0 = pa % IW
        a0 = a0 + (rows_v[bk, i0, j0, :] * rows_v[bk, j0, i0, :]
                   * (vs_s[i0] * vs_s[j0]))
        pb = pair_s[2 * k + 1]
        i1 = pb // IW
        j1 = pb % IW
        a1 = a1 + (rows_v[bk, i1, j1, :] * rows_v[bk, j1, i1, :]
                   * (vs_s[i1] * vs_s[j1]))
        return (a0, a1)

    acc0, acc1 = lax.fori_loop(
        0, npairs // 2, pk, (init, jnp.zeros((D,), jnp.float32)))
    return acc0 + acc1


def _ffm_a_body(idx_hbm, vals_hbm, emb_hbm, fw_hbm, out_hbm,
                idx_v, vals_v, rows_v, fo_v, out_v, pair_s, vs_s, sem0, sem1):
    wid = lax.axis_index("s") * NC + lax.axis_index("c")
    base = wid * B_PER_W
    pltpu.sync_copy(idx_hbm.at[pl.ds(base, B_PER_W), pl.ds(0, IW)], idx_v)
    pltpu.sync_copy(vals_hbm.at[pl.ds(base, B_PER_W), pl.ds(0, IW)], vals_v)
    _fill_pairs(pair_s, PAIRS_A)

    def fire(b, bk, sem):
        idx_row = idx_v.at[b]
        for i in range(SPLIT):
            pltpu.async_copy(emb_hbm.at[i].at[idx_row], rows_v.at[bk, i], sem)
        pltpu.async_copy(fw_hbm.at[idx_row], fo_v.at[bk], sem)

    def drain(bk, sem):
        idx_row = idx_v.at[0]
        for i in range(SPLIT):
            pltpu.make_async_copy(
                emb_hbm.at[i].at[idx_row], rows_v.at[bk, i], sem).wait()
        pltpu.make_async_copy(fw_hbm.at[idx_row], fo_v.at[bk], sem).wait()

    def compute(b, bk):
        v0, v1 = _extract_vals(vals_v, vs_s, b)
        facc = fo_v[bk, pl.ds(0, D)] * v0 + fo_v[bk, pl.ds(D, D)] * v1
        acc = _pair_sum(rows_v, pair_s, vs_s, bk, NPA, facc)
        out_v[pl.ds(b * D, D)] = acc

    fire(0, 0, sem0)

    def step(m, _):
        b0 = 2 * m
        fire(b0 + 1, 1, sem1)
        drain(0, sem0)
        compute(b0, 0)

        @pl.when(m < B_PER_W // 2 - 1)
        def _():
            fire(b0 + 2, 0, sem0)

        drain(1, sem1)
        compute(b0 + 1, 1)
        return 0

    lax.fori_loop(0, B_PER_W // 2, step, 0)
    pltpu.sync_copy(out_v, out_hbm.at[pl.ds(base * D, B_PER_W * D)])


def _ffm_b_body(idx_hbm, vals_hbm, embA_hbm, embB_hbm, acc_hbm, out_hbm,
                idx_v, vals_v, rows_v, acc_v, out_v, pair_s, vs_s, sem0, sem1):
    wid = lax.axis_index("s") * NC + lax.axis_index("c")
    base = wid * B_PER_W
    lane = _lane()
    pltpu.sync_copy(idx_hbm.at[pl.ds(base, B_PER_W), pl.ds(0, IW)], idx_v)
    pltpu.sync_copy(vals_hbm.at[pl.ds(base, B_PER_W), pl.ds(0, IW)], vals_v)
    pltpu.sync_copy(acc_hbm.at[pl.ds(base * D, B_PER_W * D)], acc_v)
    _fill_pairs(pair_s, PAIRS_B)

    def fire(b, bk, sem):
        idx_row = idx_v.at[b]
        idx_hi = idx_v.at[b, pl.ds(8, 24)]
        for i in range(SPLIT):
            pltpu.async_copy(
                embA_hbm.at[i].at[idx_hi],
                rows_v.at[bk, i, pl.ds(8, 24)], sem)
        for i in range(SPLIT, G):
            pltpu.async_copy(
                embB_hbm.at[i - SPLIT].at[idx_row], rows_v.at[bk, i], sem)

    def drain(bk, sem):
        idx_row = idx_v.at[0]
        idx_hi = idx_v.at[0, pl.ds(8, 24)]
        for i in range(SPLIT):
            pltpu.make_async_copy(
                embA_hbm.at[i].at[idx_hi],
                rows_v.at[bk, i, pl.ds(8, 24)], sem).wait()
        for i in range(SPLIT, G):
            pltpu.make_async_copy(
                embB_hbm.at[i - SPLIT].at[idx_row], rows_v.at[bk, i],
                sem).wait()

    def compute(b, bk, res):
        _extract_vals(vals_v, vs_s, b)
        facc = acc_v[pl.ds(b * D, D)]
        acc = _pair_sum(rows_v, pair_s, vs_s, bk, NPB, facc)
        for sh in (8, 4, 2, 1):
            acc = acc + acc.at[lane ^ sh].get(mode="promise_in_bounds")
        return jnp.where(lane == (b & (D - 1)), acc, res)

    fire(0, 0, sem0)

    def step(m, res):
        b0 = 2 * m
        fire(b0 + 1, 1, sem1)
        drain(0, sem0)
        res = compute(b0, 0, res)

        @pl.when(m < B_PER_W // 2 - 1)
        def _():
            fire(b0 + 2, 0, sem0)

        drain(1, sem1)
        res = compute(b0 + 1, 1, res)

        @pl.when((b0 + 1) & (D - 1) == D - 1)
        def _():
            out_v[pl.ds(b0 + 1 - (D - 1), D)] = res

        return jnp.where((b0 + 1) & (D - 1) == D - 1,
                         jnp.zeros((D,), jnp.float32), res)

    lax.fori_loop(0, B_PER_W // 2, step, jnp.zeros((D,), jnp.float32))

    def sig(k, _):
        x = out_v[pl.ds(k * D, D)]
        out_v[pl.ds(k * D, D)] = 1.0 / (1.0 + jnp.exp(-x))
        return 0

    lax.fori_loop(0, B_PER_W // D, sig, 0)
    pltpu.sync_copy(out_v, out_hbm.at[pl.ds(base, B_PER_W)])


@jax.jit
def _ffm_call(idx128, vals128, emb_tables, first_w):
    mesh = plsc.VectorSubcoreMesh(core_axis_name="c", subcore_axis_name="s")
    embA = lax.slice_in_dim(emb_tables, 0, SPLIT)
    embB = lax.slice_in_dim(emb_tables, SPLIT, G)
    fw_flat = first_w.reshape(V)
    accA = pl.kernel(
        _ffm_a_body,
        out_type=jax.ShapeDtypeStruct((B * D,), jnp.float32),
        mesh=mesh,
        compiler_params=pltpu.CompilerParams(use_tc_tiling_on_sc=False),
        scratch_types=[
            pltpu.VMEM((B_PER_W, IW), jnp.int32),
            pltpu.VMEM((B_PER_W, IW), jnp.float32),
            pltpu.VMEM((2, SPLIT, IW, D), jnp.float32),
            pltpu.VMEM((2, IW), jnp.float32),
            pltpu.VMEM((B_PER_W * D,), jnp.float32),
            pltpu.SMEM((NPA,), jnp.int32),
            pltpu.SMEM((IW,), jnp.float32),
            pltpu.SemaphoreType.DMA,
            pltpu.SemaphoreType.DMA,
        ],
    )(idx128, vals128, embA, fw_flat)
    return pl.kernel(
        _ffm_b_body,
        out_type=jax.ShapeDtypeStruct((B,), jnp.float32),
        mesh=mesh,
        compiler_params=pltpu.CompilerParams(use_tc_tiling_on_sc=False),
        scratch_types=[
            pltpu.VMEM((B_PER_W, IW), jnp.int32),
            pltpu.VMEM((B_PER_W, IW), jnp.float32),
            pltpu.VMEM((2, G, IW, D), jnp.float32),
            pltpu.VMEM((B_PER_W * D,), jnp.float32),
            pltpu.VMEM((B_PER_W,), jnp.float32),
            pltpu.SMEM((NPB,), jnp.int32),
            pltpu.SMEM((IW,), jnp.float32),
            pltpu.SemaphoreType.DMA,
            pltpu.SemaphoreType.DMA,
        ],
    )(idx128, vals128, embA, embB, accA)


def kernel(idxs, vals, emb_tables, first_w):
    idx128 = jnp.pad(idxs, ((0, 0), (0, 128 - F)))
    vals128 = jnp.pad(vals, ((0, 0), (0, 128 - F)))
    return _ffm_call(idx128, vals128, emb_tables, first_w)


# speedup vs baseline: 1.2056x; 1.2056x over previous
import jax
import jax.numpy as jnp
from jax import lax
from jax.experimental import pallas as pl
from jax.experimental.pallas import tpu as pltpu, tpu_sc as plsc

V = 100000
F = 26
D = 16
B = 4096
G = 25
NPAIR = G * (G - 1) // 2
IW = 32

NC, NS = 2, 16
NW = NC * NS
B_PER_W = B // NW


def _ffm_body(idx_hbm, vals_hbm, emb_hbm, fw_hbm, out_hbm,
              idx_v, vals_v, rows_v, fo_v, out_v, pair_s, vs_s, sem0, sem1):
    wid = lax.axis_index("s") * NC + lax.axis_index("c")
    base = wid * B_PER_W
    lane = lax.broadcasted_iota(jnp.int32, (D,), 0)

    pltpu.sync_copy(idx_hbm.at[pl.ds(base, B_PER_W), pl.ds(0, IW)], idx_v)
    pltpu.sync_copy(vals_hbm.at[pl.ds(base, B_PER_W), pl.ds(0, IW)], vals_v)

    def pt_outer(i, p):
        def pt_inner(j, p):
            pair_s[p] = i * IW + j
            return p + 1

        return lax.fori_loop(i + 1, G, pt_inner, p)

    lax.fori_loop(0, G, pt_outer, 0)

    def fire(b, bk, sem):
        idx_row = idx_v.at[b]
        for i in range(G):
            pltpu.async_copy(emb_hbm.at[i].at[idx_row], rows_v.at[bk, i], sem)
        pltpu.async_copy(fw_hbm.at[idx_row], fo_v.at[bk], sem)

    def drain(bk, sem):
        idx_row = idx_v.at[0]
        for i in range(G):
            pltpu.make_async_copy(
                emb_hbm.at[i].at[idx_row], rows_v.at[bk, i], sem
            ).wait()
        pltpu.make_async_copy(fw_hbm.at[idx_row], fo_v.at[bk], sem).wait()

    def compute(b, bk, res):
        v0 = vals_v[b, pl.ds(0, D)]
        v1 = vals_v[b, pl.ds(D, D)]
        for f in range(F):
            vs_s[f] = (v0 if f < D else v1)[f % D]
        facc = fo_v[bk, pl.ds(0, D)] * v0 + fo_v[bk, pl.ds(D, D)] * v1

        def pk(k, accs):
            a0, a1 = accs
            pa = pair_s[2 * k]
            i0 = pa // IW
            j0 = pa % IW
            a0 = a0 + (rows_v[bk, i0, j0, :] * rows_v[bk, j0, i0, :]
                       * (vs_s[i0] * vs_s[j0]))
            pb = pair_s[2 * k + 1]
            i1 = pb // IW
            j1 = pb % IW
            a1 = a1 + (rows_v[bk, i1, j1, :] * rows_v[bk, j1, i1, :]
                       * (vs_s[i1] * vs_s[j1]))
            return (a0, a1)

        acc0, acc1 = lax.fori_loop(
            0, NPAIR // 2, pk, (facc, jnp.zeros((D,), jnp.float32)))
        acc = acc0 + acc1
        for sh in (8, 4, 2, 1):
            acc = acc + acc.at[lane ^ sh].get(mode="promise_in_bounds")
        return jnp.where(lane == (b & (D - 1)), acc, res)

    fire(0, 0, sem0)

    def step(m, res):
        b0 = 2 * m
        fire(b0 + 1, 1, sem1)
        drain(0, sem0)
        res = compute(b0, 0, res)

        @pl.when(m < B_PER_W // 2 - 1)
        def _():
            fire(b0 + 2, 0, sem0)

        drain(1, sem1)
        res = compute(b0 + 1, 1, res)

        @pl.when((b0 + 1) & (D - 1) == D - 1)
        def _():
            out_v[pl.ds(b0 + 1 - (D - 1), D)] = res

        return jnp.where((b0 + 1) & (D - 1) == D - 1,
                         jnp.zeros((D,), jnp.float32), res)

    lax.fori_loop(0, B_PER_W // 2, step, jnp.zeros((D,), jnp.float32))

    def sig(k, _):
        x = out_v[pl.ds(k * D, D)]
        out_v[pl.ds(k * D, D)] = 1.0 / (1.0 + jnp.exp(-x))
        return 0

    lax.fori_loop(0, B_PER_W // D, sig, 0)
    pltpu.sync_copy(out_v, out_hbm.at[pl.ds(base, B_PER_W)])


@jax.jit
def _ffm_call(idx32, vals32, emb_tables, fw_flat):
    mesh = plsc.VectorSubcoreMesh(core_axis_name="c", subcore_axis_name="s")
    return pl.kernel(
        _ffm_body,
        out_type=jax.ShapeDtypeStruct((B,), jnp.float32),
        mesh=mesh,
        compiler_params=pltpu.CompilerParams(use_tc_tiling_on_sc=False),
        scratch_types=[
            pltpu.VMEM((B_PER_W, IW), jnp.int32),
            pltpu.VMEM((B_PER_W, IW), jnp.float32),
            pltpu.VMEM((2, G, IW, D), jnp.float32),
            pltpu.VMEM((2, IW), jnp.float32),
            pltpu.VMEM((B_PER_W,), jnp.float32),
            pltpu.SMEM((NPAIR,), jnp.int32),
            pltpu.SMEM((IW,), jnp.float32),
            pltpu.SemaphoreType.DMA,
            pltpu.SemaphoreType.DMA,
        ],
    )(idx32, vals32, emb_tables, fw_flat)


def kernel(idxs, vals, emb_tables, first_w):
    idx32 = jnp.pad(idxs, ((0, 0), (0, 128 - F)))
    vals32 = jnp.pad(vals, ((0, 0), (0, 128 - F)))
    return _ffm_call(idx32, vals32, emb_tables, first_w.reshape(V))
